# baseline (speedup 1.0000x reference)
import jax
import jax.numpy as jnp
from jax import lax
from jax.experimental import pallas as pl
from jax.experimental.pallas import tpu as pltpu

_B, _H, _W, _C = 16, 32, 32, 256
_PL = _H * _W
_SR = 2 * _C * (_PL // 128)


def _body(row_ref, col_ref, pose_ref, pemb_ref, m_ref, slab_ref):
    b = pl.program_id(0)

    @pl.when(b == 0)
    def _():
        pv = pose_ref[0:1, :]
        prow = jnp.concatenate([pv, pv], axis=1)
        pemb_ref[...] = jnp.broadcast_to(prow, (_B, 2 * _C))

        lane = lax.broadcasted_iota(jnp.int32, (_H, 128), 1)
        k = lax.broadcasted_iota(jnp.int32, (_H, 128), 0)
        cdims = (((0,), (0,)), ((), ()))

        tsel = (lane % _W == k).astype(jnp.float32)
        csl = col_ref[pl.ds(1, _W), :]
        d = lax.dot_general(csl, tsel, cdims,
                            preferred_element_type=jnp.float32)
        colE = jnp.broadcast_to(d[:, None, :], (_C, 8, 128)).reshape(_C * 8, 128)

        rsl = row_ref[pl.ds(1, _H), :]
        parts = []
        for s in range(8):
            rsel = (lane // _W + 4 * s == k).astype(jnp.float32)
            parts.append(lax.dot_general(rsl, rsel, cdims,
                                         preferred_element_type=jnp.float32))
        rowE = jnp.concatenate([p[:, None, :] for p in parts],
                               axis=1).reshape(_C * 8, 128)

        slab_ref[...] = jnp.concatenate([colE, rowE], axis=0)

    m_ref[...] = slab_ref[...][None]


def kernel(x, row_embed, col_embed, pose_token_embed):
    del x
    p_emb, m3 = pl.pallas_call(
        _body,
        grid=(_B,),
        in_specs=[
            pl.BlockSpec(row_embed.shape, lambda b: (0, 0)),
            pl.BlockSpec(col_embed.shape, lambda b: (0, 0)),
            pl.BlockSpec(pose_token_embed.shape, lambda b: (0, 0)),
        ],
        out_specs=[
            pl.BlockSpec((_B, 2 * _C), lambda b: (0, 0)),
            pl.BlockSpec((1, _SR, 128), lambda b: (b, 0, 0)),
        ],
        out_shape=[
            jax.ShapeDtypeStruct((_B, 2 * _C), jnp.float32),
            jax.ShapeDtypeStruct((_B, _SR, 128), jnp.float32),
        ],
        scratch_shapes=[pltpu.VMEM((_SR, 128), jnp.float32)],
    )(row_embed, col_embed, pose_token_embed)
    m_emb = m3.reshape(_B, 2 * _C, _H, _W)
    return (p_emb, m_emb)


# speedup vs baseline: 11.6063x; 11.6063x over previous
import jax
import jax.numpy as jnp
from jax.experimental import pallas as pl
from jax.experimental.pallas import tpu as pltpu

_B, _H, _W, _C = 16, 32, 32, 256


def _body(row_ref, col_ref, pose_ref, pemb_ref, m_ref, slab_ref):
    b = pl.program_id(0)

    @pl.when(b == 0)
    def _():
        pv = pose_ref[0:1, :]
        prow = jnp.concatenate([pv, pv], axis=1)
        pemb_ref[...] = jnp.broadcast_to(prow, (_B, 2 * _C))

        csl = col_ref[pl.ds(1, _W), :]
        rsl = row_ref[pl.ds(1, _H), :]
        colpart = jnp.broadcast_to(csl[None, :, :], (_H, _W, _C))
        rowpart = jnp.broadcast_to(rsl[:, None, :], (_H, _W, _C))
        slab_ref[...] = jnp.concatenate([colpart, rowpart], axis=2)

    m_ref[...] = slab_ref[...][None]


def kernel(x, row_embed, col_embed, pose_token_embed):
    del x
    p_emb, m4 = pl.pallas_call(
        _body,
        grid=(_B,),
        in_specs=[
            pl.BlockSpec(row_embed.shape, lambda b: (0, 0)),
            pl.BlockSpec(col_embed.shape, lambda b: (0, 0)),
            pl.BlockSpec(pose_token_embed.shape, lambda b: (0, 0)),
        ],
        out_specs=[
            pl.BlockSpec((_B, 2 * _C), lambda b: (0, 0)),
            pl.BlockSpec((1, _H, _W, 2 * _C), lambda b: (b, 0, 0, 0)),
        ],
        out_shape=[
            jax.ShapeDtypeStruct((_B, 2 * _C), jnp.float32),
            jax.ShapeDtypeStruct((_B, _H, _W, 2 * _C), jnp.float32),
        ],
        scratch_shapes=[pltpu.VMEM((_H, _W, 2 * _C), jnp.float32)],
    )(row_embed, col_embed, pose_token_embed)
    m_emb = jnp.transpose(m4, (0, 3, 1, 2))
    return (p_emb, m_emb)


# speedup vs baseline: 12.2594x; 1.0563x over previous
import jax
import jax.numpy as jnp
from jax.experimental import pallas as pl
from jax.experimental.pallas import tpu as pltpu

_B, _H, _W, _C = 16, 32, 32, 256


def _body(row_ref, col_ref, pose_ref, pemb_ref, m_ref, slab_ref, sem):
    pv = pose_ref[0:1, :]
    prow = jnp.concatenate([pv, pv], axis=1)
    pemb_ref[...] = jnp.broadcast_to(prow, (_B, 2 * _C))

    csl = col_ref[pl.ds(1, _W), :]
    rsl = row_ref[pl.ds(1, _H), :]
    colpart = jnp.broadcast_to(csl[None, :, :], (_H, _W, _C))
    rowpart = jnp.broadcast_to(rsl[:, None, :], (_H, _W, _C))
    slab_ref[...] = jnp.concatenate([colpart, rowpart], axis=2)

    copies = [pltpu.make_async_copy(slab_ref, m_ref.at[b], sem) for b in range(_B)]
    for cp in copies:
        cp.start()
    for cp in copies:
        cp.wait()


def kernel(x, row_embed, col_embed, pose_token_embed):
    del x
    p_emb, m4 = pl.pallas_call(
        _body,
        in_specs=[
            pl.BlockSpec(memory_space=pltpu.VMEM),
            pl.BlockSpec(memory_space=pltpu.VMEM),
            pl.BlockSpec(memory_space=pltpu.VMEM),
        ],
        out_specs=[
            pl.BlockSpec(memory_space=pltpu.VMEM),
            pl.BlockSpec(memory_space=pl.MemorySpace.ANY),
        ],
        out_shape=[
            jax.ShapeDtypeStruct((_B, 2 * _C), jnp.float32),
            jax.ShapeDtypeStruct((_B, _H, _W, 2 * _C), jnp.float32),
        ],
        scratch_shapes=[
            pltpu.VMEM((_H, _W, 2 * _C), jnp.float32),
            pltpu.SemaphoreType.DMA,
        ],
    )(row_embed, col_embed, pose_token_embed)
    m_emb = jnp.transpose(m4, (0, 3, 1, 2))
    return (p_emb, m_emb)
